# baseline (speedup 1.0000x reference)
import functools

import jax
import jax.numpy as jnp
from jax import lax
from jax.experimental import pallas as pl
from jax.experimental.pallas import tpu as pltpu
from jax.experimental.pallas import tpu_sc as plsc

N_NODES = 10000
N_PAD = 10240
E_EDGES = 320000
D_IN = 128
HID = 16
N_CLS = 40

NC = 2
NS = 16
NW = NC * NS
EPW = E_EDGES // NW
SUB = 16
SUBE = 125
CHUNK = SUB * SUBE
NCHUNK = EPW // CHUNK
RPT = N_PAD // NS

_mesh = plsc.VectorSubcoreMesh(core_axis_name="c", subcore_axis_name="s")


def _rsqrt16(d):
    i = lax.bitcast_convert_type(d, jnp.int32)
    i = 0x5F3759DF - lax.shift_right_logical(i, 1)
    y = lax.bitcast_convert_type(i, jnp.float32)
    for _ in range(3):
        y = y * (1.5 - 0.5 * d * y * y)
    return y


EROWS = E_EDGES // SUBE
DROWS = EROWS // NS
NCHUNK_DEG = DROWS // SUB


def _make_edge_pass(layer):
    extra_out = jax.ShapeDtypeStruct(
        (N_PAD,) if layer == 1 else (N_PAD, HID), jnp.float32)
    extra_scratch = [
        pltpu.VMEM((2, SUB, SUBE), jnp.float32),
        pltpu.VMEM_SHARED((N_PAD,), jnp.float32),
    ] if layer == 1 else [
        pltpu.VMEM((RPT, HID), jnp.float32),
        pltpu.VMEM((RPT, HID), jnp.float32),
        pltpu.VMEM((16,), jnp.float32),
    ]

    @functools.partial(
        pl.kernel,
        mesh=_mesh,
        compiler_params=pltpu.CompilerParams(use_tc_tiling_on_sc=False),
        out_type=(
            jax.ShapeDtypeStruct((NC, N_PAD, HID), jnp.float32),
            extra_out,
        ),
        scratch_types=[
            pltpu.VMEM((2, SUB, SUBE), jnp.int32),
            pltpu.VMEM((2, SUB, SUBE), jnp.int32),
            pltpu.VMEM((2, CHUNK), jnp.float32),
            pltpu.VMEM((2, CHUNK, HID), jnp.float32),
            pltpu.VMEM((RPT,), jnp.float32),
            pltpu.VMEM((RPT, HID), jnp.float32),
            pltpu.VMEM_SHARED((N_PAD, HID), jnp.float32),
            pltpu.VMEM_SHARED((N_PAD, HID), jnp.float32),
            pltpu.SemaphoreType.DMA,
            pltpu.SemaphoreType.DMA,
            pltpu.SemaphoreType.DMA,
            pltpu.SemaphoreType.DMA,
        ] + extra_scratch,
    )
    def _pass(*args):
        if layer == 1:
            (row_hbm, col_hbm, ewf_hbm, ew2_hbm, h1_hbm,
             acc_out, extra_hbm,
             ridx, cidx, ewf, buf, dv, hs, table, acc,
             gs0, gs1, ss0, ss1, ewd, dacc) = args
        else:
            (row_hbm, col_hbm, ewf_hbm, dis_hbm, h1_hbm, acc1_hbm, b1_hbm,
             acc_out, extra_hbm,
             ridx, cidx, ewf, buf, dv, hs, table, acc,
             gs0, gs1, ss0, ss1, a0v, a1v, b1v) = args
        c = lax.axis_index("c")
        s = lax.axis_index("s")
        wid = s * NC + c
        gsem = (gs0, gs1)
        ssem = (ss0, ss1)
        base = s * RPT

        pltpu.sync_copy(h1_hbm.at[pl.ds(base, RPT)], hs)
        if layer == 1:
            def _dz(i, _):
                dv[pl.ds(i * 16, 16)] = jnp.zeros((16,), jnp.float32)
                return 0

            lax.fori_loop(0, RPT // 16, _dz, 0)
            pltpu.sync_copy(dv, dacc.at[pl.ds(base, RPT)])
            plsc.subcore_barrier()

            def _dload(t):
                p = t % 2
                db = s * DROWS + t * SUB
                pltpu.sync_copy(col_hbm.at[pl.ds(db, SUB)], cidx.at[p])
                pltpu.sync_copy(ew2_hbm.at[pl.ds(db, SUB)], ewd.at[p])

            def _dfire(t):
                p = t % 2
                return [
                    pltpu.async_copy(
                        ewd.at[p, j],
                        dacc.at[cidx.at[p, j]], ssem[p], add=True)
                    for j in range(SUB)
                ]

            _dload(0)
            dpend = {0: _dfire(0)}
            for t in range(1, NCHUNK_DEG):
                if t - 2 in dpend:
                    for d in dpend.pop(t - 2):
                        d.wait()
                _dload(t)
                dpend[t] = _dfire(t)
            for t in sorted(dpend):
                for d in dpend.pop(t):
                    d.wait()
            plsc.subcore_barrier()
            pltpu.sync_copy(dacc.at[pl.ds(base, RPT)], dv)

            def _dis(i, _):
                d = dv[pl.ds(i * 16, 16)] + 1.0
                dv[pl.ds(i * 16, 16)] = _rsqrt16(d)
                return 0

            lax.fori_loop(0, RPT // 16, _dis, 0)

            @pl.when(c == 0)
            def _emit_dis():
                pltpu.sync_copy(dv, extra_hbm.at[pl.ds(base, RPT)])

            def _grow(g, _):
                w = dv[pl.ds(g * 16, 16)]
                for k in range(16):
                    hs[g * 16 + k] = hs[g * 16 + k] * w[k]
                return 0
        else:
            pltpu.sync_copy(dis_hbm.at[pl.ds(base, RPT)], dv)
            pltpu.sync_copy(acc1_hbm.at[0, pl.ds(base, RPT)], a0v)
            pltpu.sync_copy(acc1_hbm.at[1, pl.ds(base, RPT)], a1v)
            pltpu.sync_copy(b1_hbm, b1v)
            b1row = b1v[...]

            def _grow(g, _):
                w = dv[pl.ds(g * 16, 16)]
                for k in range(16):
                    r = g * 16 + k
                    x1 = w[k] * (a0v[r] + a1v[r]) \
                        + (w[k] * w[k]) * hs[r] + b1row
                    a0v[r] = x1
                    hs[r] = w[k] * x1
                return 0

        lax.fori_loop(0, RPT // 16, _grow, 0)
        if layer == 2:
            pltpu.sync_copy(a0v, extra_hbm.at[pl.ds(base, RPT)])
        pltpu.sync_copy(hs, table.at[pl.ds(base, RPT)])

        def _zrow(i, _):
            hs[i] = jnp.zeros((HID,), jnp.float32)
            return 0

        lax.fori_loop(0, RPT, _zrow, 0)
        pltpu.sync_copy(hs, acc.at[pl.ds(base, RPT)])
        plsc.subcore_barrier()

        def _idx_load(t):
            p = t % 2
            b2 = wid * (EPW // SUBE) + t * SUB
            pltpu.sync_copy(row_hbm.at[pl.ds(b2, SUB)], ridx.at[p])
            pltpu.sync_copy(col_hbm.at[pl.ds(b2, SUB)], cidx.at[p])
            pltpu.sync_copy(ewf_hbm.at[pl.ds(wid * EPW + t * CHUNK, CHUNK)],
                            ewf.at[p])

        def _fire_gathers(t):
            p = t % 2
            return [
                pltpu.async_copy(table.at[ridx.at[p, j]],
                                 buf.at[p, pl.ds(j * SUBE, SUBE)], gsem[p])
                for j in range(SUB)
            ]

        def _fire_scatters(t):
            p = t % 2
            return [
                pltpu.async_copy(buf.at[p, pl.ds(j * SUBE, SUBE)],
                                 acc.at[cidx.at[p, j]], ssem[p], add=True)
                for j in range(SUB)
            ]

        def _scale(t):
            p = t % 2

            def _grp(g, _):
                w = ewf[p, pl.ds(g * 16, 16)]
                for k in range(16):
                    buf[p, g * 16 + k] = buf[p, g * 16 + k] * w[k]
                return 0

            lax.fori_loop(0, CHUNK // 16, _grp, 0)

        _idx_load(0)
        g_pend = {0: _fire_gathers(0)}
        s_pend = {}
        for t in range(NCHUNK):
            if t + 1 < NCHUNK:
                if t - 1 in s_pend:
                    for d in s_pend.pop(t - 1):
                        d.wait()
                _idx_load(t + 1)
            for d in g_pend.pop(t):
                d.wait()
            if t + 1 < NCHUNK:
                g_pend[t + 1] = _fire_gathers(t + 1)
            _scale(t)
            s_pend[t] = _fire_scatters(t)
        for t in sorted(s_pend):
            for d in s_pend.pop(t):
                d.wait()
        plsc.subcore_barrier()
        pltpu.sync_copy(acc.at[pl.ds(base, RPT)],
                        acc_out.at[c, pl.ds(base, RPT)])

    return _pass


_edge_pass1 = _make_edge_pass(1)
_edge_pass2 = _make_edge_pass(2)


def _mm1_body(x_ref, w_ref, o_ref):
    o_ref[...] = jnp.dot(x_ref[...], w_ref[...],
                         preferred_element_type=jnp.float32)


def _final_body(a0_ref, a1_ref, x1_ref, dis_ref, w2_ref, b2_ref, o_ref):
    dis = dis_ref[...]
    agg = dis * (a0_ref[...] + a1_ref[...]) + (dis * dis) * x1_ref[...]
    x2 = jnp.dot(agg, w2_ref[...], preferred_element_type=jnp.float32) \
        + b2_ref[...]
    m = jnp.max(x2, axis=1, keepdims=True)
    e = jnp.exp(x2 - m)
    lse = jnp.log(jnp.sum(e, axis=1, keepdims=True))
    o_ref[...] = x2 - m - lse


def kernel(x, edge_index, edge_weight, W1, b1, W2, b2):
    row_r = edge_index[0].reshape(E_EDGES // SUBE, SUBE)
    col_r = edge_index[1].reshape(E_EDGES // SUBE, SUBE)
    x_pad = jnp.pad(x, ((0, N_PAD - N_NODES), (0, 0)))

    h1 = pl.pallas_call(
        _mm1_body,
        grid=(NS,),
        in_specs=[
            pl.BlockSpec((RPT, D_IN), lambda i: (i, 0)),
            pl.BlockSpec((D_IN, HID), lambda i: (0, 0)),
        ],
        out_specs=pl.BlockSpec((RPT, HID), lambda i: (i, 0)),
        out_shape=jax.ShapeDtypeStruct((N_PAD, HID), jnp.float32),
    )(x_pad, W1)

    ew_r = edge_weight.reshape(E_EDGES // SUBE, SUBE)
    acc1, dis = _edge_pass1(row_r, col_r, edge_weight, ew_r, h1)

    acc2, x1p = _edge_pass2(row_r, col_r, edge_weight, dis, h1, acc1, b1)

    out = pl.pallas_call(
        _final_body,
        out_shape=jax.ShapeDtypeStruct((N_PAD, N_CLS), jnp.float32),
    )(acc2[0], acc2[1], x1p, dis[:, None], W2, b2[None, :])

    return (out[:N_NODES], x1p[:N_NODES])


# speedup vs baseline: 1.2724x; 1.2724x over previous
import functools

import jax
import jax.numpy as jnp
from jax import lax
from jax.experimental import pallas as pl
from jax.experimental.pallas import tpu as pltpu
from jax.experimental.pallas import tpu_sc as plsc

N_NODES = 10000
N_PAD = 10240
E_EDGES = 320000
D_IN = 128
HID = 16
N_CLS = 40

NC = 2
NS = 16
NW = NC * NS
EPW = E_EDGES // NW
SUB = 16
SUBE = 125
CHUNK = SUB * SUBE
NCHUNK = EPW // CHUNK
RPT = N_PAD // NS

_mesh = plsc.VectorSubcoreMesh(core_axis_name="c", subcore_axis_name="s")


def _rsqrt16(d):
    i = lax.bitcast_convert_type(d, jnp.int32)
    i = 0x5F3759DF - lax.shift_right_logical(i, 1)
    y = lax.bitcast_convert_type(i, jnp.float32)
    for _ in range(3):
        y = y * (1.5 - 0.5 * d * y * y)
    return y


@functools.partial(
    pl.kernel,
    mesh=_mesh,
    compiler_params=pltpu.CompilerParams(use_tc_tiling_on_sc=False),
    out_type=jax.ShapeDtypeStruct((NC, N_PAD), jnp.float32),
    scratch_types=[
        pltpu.VMEM((2, SUB, SUBE), jnp.int32),
        pltpu.VMEM((2, SUB, SUBE), jnp.float32),
        pltpu.VMEM((RPT,), jnp.float32),
        pltpu.VMEM_SHARED((N_PAD,), jnp.float32),
        pltpu.SemaphoreType.DMA,
        pltpu.SemaphoreType.DMA,
    ],
)
def _deg_pass(ei_hbm, ew_hbm, out_hbm, cidx, ewv, stage, acc, ds0, ds1):
    c = lax.axis_index("c")
    s = lax.axis_index("s")
    wid = s * NC + c
    dsem = (ds0, ds1)

    def _zero(i, _):
        stage[pl.ds(i * 16, 16)] = jnp.zeros((16,), jnp.float32)
        return 0

    lax.fori_loop(0, RPT // 16, _zero, 0)
    pltpu.sync_copy(stage, acc.at[pl.ds(s * RPT, RPT)])
    plsc.subcore_barrier()

    def _load(t):
        p = t % 2
        base = wid * (EPW // SUBE) + t * SUB
        pltpu.sync_copy(ei_hbm.at[1, pl.ds(base, SUB)], cidx.at[p])
        pltpu.sync_copy(ew_hbm.at[pl.ds(base, SUB)], ewv.at[p])

    def _fire(t):
        p = t % 2
        return [
            pltpu.async_copy(ewv.at[p, j], acc.at[cidx.at[p, j]],
                             dsem[p], add=True)
            for j in range(SUB)
        ]

    _load(0)
    pend = {0: _fire(0)}
    for t in range(1, NCHUNK):
        if t - 2 in pend:
            for d in pend.pop(t - 2):
                d.wait()
        _load(t)
        pend[t] = _fire(t)
    for t in sorted(pend):
        for d in pend.pop(t):
            d.wait()
    plsc.subcore_barrier()
    pltpu.sync_copy(acc.at[pl.ds(s * RPT, RPT)],
                    out_hbm.at[c, pl.ds(s * RPT, RPT)])


def _make_edge_pass(layer):
    extra_out = jax.ShapeDtypeStruct(
        (N_PAD,) if layer == 1 else (N_PAD, HID), jnp.float32)
    extra_scratch = [] if layer == 1 else [
        pltpu.VMEM((RPT, HID), jnp.float32),
        pltpu.VMEM((RPT, HID), jnp.float32),
        pltpu.VMEM((16,), jnp.float32),
    ]

    @functools.partial(
        pl.kernel,
        mesh=_mesh,
        compiler_params=pltpu.CompilerParams(use_tc_tiling_on_sc=False),
        out_type=(
            jax.ShapeDtypeStruct((NC, N_PAD, HID), jnp.float32),
            extra_out,
        ),
        scratch_types=[
            pltpu.VMEM((2, SUB, SUBE), jnp.int32),
            pltpu.VMEM((2, SUB, SUBE), jnp.int32),
            pltpu.VMEM((2, CHUNK), jnp.float32),
            pltpu.VMEM((2, CHUNK, HID), jnp.float32),
            pltpu.VMEM((RPT,), jnp.float32),
            pltpu.VMEM((RPT,), jnp.float32),
            pltpu.VMEM((RPT, HID), jnp.float32),
            pltpu.VMEM_SHARED((N_PAD, HID), jnp.float32),
            pltpu.VMEM_SHARED((N_PAD, HID), jnp.float32),
            pltpu.SemaphoreType.DMA,
            pltpu.SemaphoreType.DMA,
            pltpu.SemaphoreType.DMA,
            pltpu.SemaphoreType.DMA,
        ] + extra_scratch,
    )
    def _pass(*args):
        if layer == 1:
            (ei_hbm, ewf_hbm, deg_hbm, h1_hbm,
             acc_out, extra_hbm,
             ridx, cidx, ewf, buf, dv, d1v, hs, table, acc,
             gs0, gs1, ss0, ss1) = args
        else:
            (ei_hbm, ewf_hbm, deg_hbm, h1_hbm, acc1_hbm, b1_hbm,
             acc_out, extra_hbm,
             ridx, cidx, ewf, buf, dv, d1v, hs, table, acc,
             gs0, gs1, ss0, ss1, a0v, a1v, b1v) = args
        c = lax.axis_index("c")
        s = lax.axis_index("s")
        wid = s * NC + c
        gsem = (gs0, gs1)
        ssem = (ss0, ss1)
        base = s * RPT

        pltpu.sync_copy(deg_hbm.at[0, pl.ds(base, RPT)], dv)
        pltpu.sync_copy(deg_hbm.at[1, pl.ds(base, RPT)], d1v)
        pltpu.sync_copy(h1_hbm.at[pl.ds(base, RPT)], hs)
        if layer == 2:
            pltpu.sync_copy(acc1_hbm.at[0, pl.ds(base, RPT)], a0v)
            pltpu.sync_copy(acc1_hbm.at[1, pl.ds(base, RPT)], a1v)
            pltpu.sync_copy(b1_hbm, b1v)

        def _dis(i, _):
            d = dv[pl.ds(i * 16, 16)] + d1v[pl.ds(i * 16, 16)] + 1.0
            dv[pl.ds(i * 16, 16)] = _rsqrt16(d)
            return 0

        lax.fori_loop(0, RPT // 16, _dis, 0)
        if layer == 1:
            pltpu.sync_copy(dv, extra_hbm.at[pl.ds(base, RPT)])

            def _grow(g, _):
                w = dv[pl.ds(g * 16, 16)]
                for k in range(16):
                    hs[g * 16 + k] = hs[g * 16 + k] * w[k]
                return 0
        else:
            b1row = b1v[...]

            def _grow(g, _):
                w = dv[pl.ds(g * 16, 16)]
                for k in range(16):
                    r = g * 16 + k
                    x1 = w[k] * (a0v[r] + a1v[r]) \
                        + (w[k] * w[k]) * hs[r] + b1row
                    a0v[r] = x1
                    hs[r] = w[k] * x1
                return 0

        lax.fori_loop(0, RPT // 16, _grow, 0)
        if layer == 2:
            pltpu.sync_copy(a0v, extra_hbm.at[pl.ds(base, RPT)])
        pltpu.sync_copy(hs, table.at[pl.ds(base, RPT)])

        def _zrow(i, _):
            hs[i] = jnp.zeros((HID,), jnp.float32)
            return 0

        lax.fori_loop(0, RPT, _zrow, 0)
        pltpu.sync_copy(hs, acc.at[pl.ds(base, RPT)])
        plsc.subcore_barrier()

        def _idx_load(t):
            p = t % 2
            b2 = wid * (EPW // SUBE) + t * SUB
            pltpu.sync_copy(ei_hbm.at[0, pl.ds(b2, SUB)], ridx.at[p])
            pltpu.sync_copy(ei_hbm.at[1, pl.ds(b2, SUB)], cidx.at[p])
            pltpu.sync_copy(ewf_hbm.at[pl.ds(wid * EPW + t * CHUNK, CHUNK)],
                            ewf.at[p])

        def _fire_gathers(t):
            p = t % 2
            return [
                pltpu.async_copy(table.at[ridx.at[p, j]],
                                 buf.at[p, pl.ds(j * SUBE, SUBE)], gsem[p])
                for j in range(SUB)
            ]

        def _fire_scatters(t):
            p = t % 2
            return [
                pltpu.async_copy(buf.at[p, pl.ds(j * SUBE, SUBE)],
                                 acc.at[cidx.at[p, j]], ssem[p], add=True)
                for j in range(SUB)
            ]

        def _scale(t):
            p = t % 2

            @plsc.parallel_loop(0, CHUNK // 16, unroll=2)
            def _grp(g):
                w = ewf[p, pl.ds(g * 16, 16)]
                for k in range(16):
                    buf[p, g * 16 + k] = buf[p, g * 16 + k] * w[k]

        _idx_load(0)
        g_pend = {0: _fire_gathers(0)}
        s_pend = {}
        for t in range(NCHUNK):
            if t + 1 < NCHUNK:
                if t - 1 in s_pend:
                    for d in s_pend.pop(t - 1):
                        d.wait()
                _idx_load(t + 1)
            for d in g_pend.pop(t):
                d.wait()
            if t + 1 < NCHUNK:
                g_pend[t + 1] = _fire_gathers(t + 1)
            _scale(t)
            s_pend[t] = _fire_scatters(t)
        for t in sorted(s_pend):
            for d in s_pend.pop(t):
                d.wait()
        plsc.subcore_barrier()
        pltpu.sync_copy(acc.at[pl.ds(base, RPT)],
                        acc_out.at[c, pl.ds(base, RPT)])

    return _pass


_edge_pass1 = _make_edge_pass(1)
_edge_pass2 = _make_edge_pass(2)


def _mm1_body(x_ref, w_ref, o_ref):
    o_ref[pl.ds(0, N_NODES), :] = jnp.dot(
        x_ref[...], w_ref[...], preferred_element_type=jnp.float32)
    o_ref[pl.ds(N_NODES, N_PAD - N_NODES), :] = jnp.zeros(
        (N_PAD - N_NODES, HID), jnp.float32)


def _final_body(acc2_ref, x1p_ref, dis_ref, w2_ref, b2_ref,
                o_ref, x1_ref):
    dis = dis_ref[...]
    x1 = x1p_ref[...]
    x1_ref[...] = x1
    agg = dis * (acc2_ref[0] + acc2_ref[1]) + (dis * dis) * x1
    x2 = jnp.dot(agg, w2_ref[...], preferred_element_type=jnp.float32) \
        + b2_ref[...]
    m = jnp.max(x2, axis=1, keepdims=True)
    e = jnp.exp(x2 - m)
    lse = jnp.log(jnp.sum(e, axis=1, keepdims=True))
    o_ref[...] = x2 - m - lse


_FBLK = 2000


def kernel(x, edge_index, edge_weight, W1, b1, W2, b2):
    ei_r = edge_index.reshape(2, E_EDGES // SUBE, SUBE)
    ew_r = edge_weight.reshape(E_EDGES // SUBE, SUBE)

    h1 = pl.pallas_call(
        _mm1_body,
        out_shape=jax.ShapeDtypeStruct((N_PAD, HID), jnp.float32),
    )(x, W1)

    deg_parts = _deg_pass(ei_r, ew_r)

    acc1, dis = _edge_pass1(ei_r, edge_weight, deg_parts, h1)

    acc2, x1p = _edge_pass2(ei_r, edge_weight, deg_parts, h1, acc1, b1)

    out, x1 = pl.pallas_call(
        _final_body,
        grid=(N_NODES // _FBLK,),
        in_specs=[
            pl.BlockSpec((NC, _FBLK, HID), lambda i: (0, i, 0)),
            pl.BlockSpec((_FBLK, HID), lambda i: (i, 0)),
            pl.BlockSpec((_FBLK, 1), lambda i: (i, 0)),
            pl.BlockSpec((HID, N_CLS), lambda i: (0, 0)),
            pl.BlockSpec((1, N_CLS), lambda i: (0, 0)),
        ],
        out_specs=(
            pl.BlockSpec((_FBLK, N_CLS), lambda i: (i, 0)),
            pl.BlockSpec((_FBLK, HID), lambda i: (i, 0)),
        ),
        out_shape=(
            jax.ShapeDtypeStruct((N_NODES, N_CLS), jnp.float32),
            jax.ShapeDtypeStruct((N_NODES, HID), jnp.float32),
        ),
    )(acc2, x1p, dis[:, None], W2, b2[None, :])

    return (out, x1)


# speedup vs baseline: 1.3093x; 1.0290x over previous
import functools

import jax
import jax.numpy as jnp
from jax import lax
from jax.experimental import pallas as pl
from jax.experimental.pallas import tpu as pltpu
from jax.experimental.pallas import tpu_sc as plsc

N_NODES = 10000
N_PAD = 10240
E_EDGES = 320000
E_PAD = 327680
D_IN = 128
HID = 16
N_CLS = 40

NC = 2
NS = 16
NW = NC * NS
EPW = E_PAD // NW
SUB = 16
SUBE = 128
CHUNK = SUB * SUBE
NCHUNK = EPW // CHUNK
RPT = N_PAD // NS

_mesh = plsc.VectorSubcoreMesh(core_axis_name="c", subcore_axis_name="s")


def _rsqrt16(d):
    i = lax.bitcast_convert_type(d, jnp.int32)
    i = 0x5F3759DF - lax.shift_right_logical(i, 1)
    y = lax.bitcast_convert_type(i, jnp.float32)
    for _ in range(3):
        y = y * (1.5 - 0.5 * d * y * y)
    return y


@functools.partial(
    pl.kernel,
    mesh=_mesh,
    compiler_params=pltpu.CompilerParams(use_tc_tiling_on_sc=False),
    out_type=jax.ShapeDtypeStruct((NC, N_PAD), jnp.float32),
    scratch_types=[
        pltpu.VMEM((2, SUB, SUBE), jnp.int32),
        pltpu.VMEM((2, SUB, SUBE), jnp.float32),
        pltpu.VMEM((RPT,), jnp.float32),
        pltpu.VMEM_SHARED((N_PAD,), jnp.float32),
        pltpu.SemaphoreType.DMA,
        pltpu.SemaphoreType.DMA,
        pltpu.SemaphoreType.DMA,
        pltpu.SemaphoreType.DMA,
    ],
)
def _deg_pass(ei_hbm, ew_hbm, out_hbm, cidx, ewv, stage, acc,
              ds0, ds1, is0, is1):
    c = lax.axis_index("c")
    s = lax.axis_index("s")
    wid = s * NC + c
    dsem = (ds0, ds1)
    isem = (is0, is1)

    def _zero(i, _):
        stage[pl.ds(i * 16, 16)] = jnp.zeros((16,), jnp.float32)
        return 0

    lax.fori_loop(0, RPT // 16, _zero, 0)
    pltpu.sync_copy(stage, acc.at[pl.ds(s * RPT, RPT)])
    plsc.subcore_barrier()

    def _load(t):
        p = t % 2
        base = wid * (EPW // SUBE) + t * SUB
        return [
            pltpu.async_copy(ei_hbm.at[1, pl.ds(base, SUB)], cidx.at[p],
                             isem[p]),
            pltpu.async_copy(ew_hbm.at[pl.ds(base, SUB)], ewv.at[p],
                             isem[p]),
        ]

    def _fire(t):
        p = t % 2
        return [
            pltpu.async_copy(ewv.at[p, j], acc.at[cidx.at[p, j]],
                             dsem[p], add=True)
            for j in range(SUB)
        ]

    i_pend = {0: _load(0)}
    pend = {}
    for t in range(NCHUNK):
        for d in i_pend.pop(t):
            d.wait()
        if t + 1 < NCHUNK:
            if t - 1 in pend:
                for d in pend.pop(t - 1):
                    d.wait()
            i_pend[t + 1] = _load(t + 1)
        pend[t] = _fire(t)
    for t in sorted(pend):
        for d in pend.pop(t):
            d.wait()
    plsc.subcore_barrier()
    pltpu.sync_copy(acc.at[pl.ds(s * RPT, RPT)],
                    out_hbm.at[c, pl.ds(s * RPT, RPT)])


def _make_edge_pass(layer):
    extra_out = jax.ShapeDtypeStruct(
        (N_PAD,) if layer == 1 else (N_PAD, HID), jnp.float32)
    extra_scratch = [] if layer == 1 else [
        pltpu.VMEM((RPT, HID), jnp.float32),
        pltpu.VMEM((RPT, HID), jnp.float32),
        pltpu.VMEM((16,), jnp.float32),
    ]

    @functools.partial(
        pl.kernel,
        mesh=_mesh,
        compiler_params=pltpu.CompilerParams(use_tc_tiling_on_sc=False),
        out_type=(
            jax.ShapeDtypeStruct((NC, N_PAD, HID), jnp.float32),
            extra_out,
        ),
        scratch_types=[
            pltpu.VMEM((2, SUB, SUBE), jnp.int32),
            pltpu.VMEM((2, SUB, SUBE), jnp.int32),
            pltpu.VMEM((2, CHUNK), jnp.float32),
            pltpu.VMEM((2, CHUNK, HID), jnp.float32),
            pltpu.VMEM((RPT,), jnp.float32),
            pltpu.VMEM((RPT,), jnp.float32),
            pltpu.VMEM((RPT, HID), jnp.float32),
            pltpu.VMEM_SHARED((N_PAD, HID), jnp.float32),
            pltpu.VMEM_SHARED((N_PAD, HID), jnp.float32),
            pltpu.SemaphoreType.DMA,
            pltpu.SemaphoreType.DMA,
            pltpu.SemaphoreType.DMA,
            pltpu.SemaphoreType.DMA,
            pltpu.SemaphoreType.DMA,
            pltpu.SemaphoreType.DMA,
        ] + extra_scratch,
    )
    def _pass(*args):
        if layer == 1:
            (ei_hbm, ewf_hbm, deg_hbm, h1_hbm,
             acc_out, extra_hbm,
             ridx, cidx, ewf, buf, dv, d1v, hs, table, acc,
             gs0, gs1, ss0, ss1, is0, is1) = args
        else:
            (ei_hbm, ewf_hbm, deg_hbm, h1_hbm, acc1_hbm, b1_hbm,
             acc_out, extra_hbm,
             ridx, cidx, ewf, buf, dv, d1v, hs, table, acc,
             gs0, gs1, ss0, ss1, is0, is1, a0v, a1v, b1v) = args
        c = lax.axis_index("c")
        s = lax.axis_index("s")
        wid = s * NC + c
        gsem = (gs0, gs1)
        ssem = (ss0, ss1)
        isem = (is0, is1)
        base = s * RPT

        pltpu.sync_copy(deg_hbm.at[0, pl.ds(base, RPT)], dv)
        pltpu.sync_copy(deg_hbm.at[1, pl.ds(base, RPT)], d1v)
        pltpu.sync_copy(h1_hbm.at[pl.ds(base, RPT)], hs)
        if layer == 2:
            pltpu.sync_copy(acc1_hbm.at[0, pl.ds(base, RPT)], a0v)
            pltpu.sync_copy(acc1_hbm.at[1, pl.ds(base, RPT)], a1v)
            pltpu.sync_copy(b1_hbm, b1v)

        def _dis(i, _):
            d = dv[pl.ds(i * 16, 16)] + d1v[pl.ds(i * 16, 16)] + 1.0
            dv[pl.ds(i * 16, 16)] = _rsqrt16(d)
            return 0

        lax.fori_loop(0, RPT // 16, _dis, 0)
        if layer == 1:
            pltpu.sync_copy(dv, extra_hbm.at[pl.ds(base, RPT)])

            def _grow(g, _):
                w = dv[pl.ds(g * 16, 16)]
                for k in range(16):
                    hs[g * 16 + k] = hs[g * 16 + k] * w[k]
                return 0
        else:
            b1row = b1v[...]

            def _grow(g, _):
                w = dv[pl.ds(g * 16, 16)]
                for k in range(16):
                    r = g * 16 + k
                    x1 = w[k] * (a0v[r] + a1v[r]) \
                        + (w[k] * w[k]) * hs[r] + b1row
                    a0v[r] = x1
                    hs[r] = w[k] * x1
                return 0

        lax.fori_loop(0, RPT // 16, _grow, 0)
        if layer == 2:
            pltpu.sync_copy(a0v, extra_hbm.at[pl.ds(base, RPT)])
        pltpu.sync_copy(hs, table.at[pl.ds(base, RPT)])

        def _zrow(i, _):
            hs[i] = jnp.zeros((HID,), jnp.float32)
            return 0

        lax.fori_loop(0, RPT, _zrow, 0)
        pltpu.sync_copy(hs, acc.at[pl.ds(base, RPT)])
        plsc.subcore_barrier()

        def _idx_fire(t):
            p = t % 2
            b2 = wid * (EPW // SUBE) + t * SUB
            return [
                pltpu.async_copy(ei_hbm.at[0, pl.ds(b2, SUB)], ridx.at[p],
                                 isem[p]),
                pltpu.async_copy(ei_hbm.at[1, pl.ds(b2, SUB)], cidx.at[p],
                                 isem[p]),
                pltpu.async_copy(
                    ewf_hbm.at[pl.ds(wid * EPW + t * CHUNK, CHUNK)],
                    ewf.at[p], isem[p]),
            ]

        def _fire_gathers(t):
            p = t % 2
            return [
                pltpu.async_copy(table.at[ridx.at[p, j]],
                                 buf.at[p, pl.ds(j * SUBE, SUBE)], gsem[p])
                for j in range(SUB)
            ]

        def _fire_scatters(t):
            p = t % 2
            return [
                pltpu.async_copy(buf.at[p, pl.ds(j * SUBE, SUBE)],
                                 acc.at[cidx.at[p, j]], ssem[p], add=True)
                for j in range(SUB)
            ]

        def _scale(t):
            p = t % 2

            @plsc.parallel_loop(0, CHUNK // 16, unroll=2)
            def _grp(g):
                w = ewf[p, pl.ds(g * 16, 16)]
                for k in range(16):
                    buf[p, g * 16 + k] = buf[p, g * 16 + k] * w[k]

        i_pend = {0: _idx_fire(0)}
        for d in i_pend.pop(0):
            d.wait()
        g_pend = {0: _fire_gathers(0)}
        s_pend = {}
        for t in range(NCHUNK):
            if t + 1 < NCHUNK:
                if t - 1 in s_pend:
                    for d in s_pend.pop(t - 1):
                        d.wait()
                i_pend[t + 1] = _idx_fire(t + 1)
            for d in g_pend.pop(t):
                d.wait()
            if t + 1 < NCHUNK:
                for d in i_pend.pop(t + 1):
                    d.wait()
                g_pend[t + 1] = _fire_gathers(t + 1)
            _scale(t)
            s_pend[t] = _fire_scatters(t)
        for t in sorted(s_pend):
            for d in s_pend.pop(t):
                d.wait()
        plsc.subcore_barrier()
        pltpu.sync_copy(acc.at[pl.ds(base, RPT)],
                        acc_out.at[c, pl.ds(base, RPT)])

    return _pass


_edge_pass1 = _make_edge_pass(1)
_edge_pass2 = _make_edge_pass(2)


def _mm1_body(x_ref, w_ref, o_ref):
    o_ref[pl.ds(0, N_NODES), :] = jnp.dot(
        x_ref[...], w_ref[...], preferred_element_type=jnp.float32)
    o_ref[pl.ds(N_NODES, N_PAD - N_NODES), :] = jnp.zeros(
        (N_PAD - N_NODES, HID), jnp.float32)


def _final_body(acc2_ref, x1p_ref, dis_ref, w2_ref, b2_ref,
                o_ref, x1_ref):
    dis = dis_ref[...]
    x1 = x1p_ref[...]
    x1_ref[...] = x1
    agg = dis * (acc2_ref[0] + acc2_ref[1]) + (dis * dis) * x1
    x2 = jnp.dot(agg, w2_ref[...], preferred_element_type=jnp.float32) \
        + b2_ref[...]
    m = jnp.max(x2, axis=1, keepdims=True)
    e = jnp.exp(x2 - m)
    lse = jnp.log(jnp.sum(e, axis=1, keepdims=True))
    o_ref[...] = x2 - m - lse


_FBLK = 2000


def kernel(x, edge_index, edge_weight, W1, b1, W2, b2):
    ei_p = jnp.pad(edge_index, ((0, 0), (0, E_PAD - E_EDGES)))
    ew_p = jnp.pad(edge_weight, (0, E_PAD - E_EDGES))
    ei_r = ei_p.reshape(2, E_PAD // SUBE, SUBE)
    ew_r = ew_p.reshape(E_PAD // SUBE, SUBE)

    h1 = pl.pallas_call(
        _mm1_body,
        out_shape=jax.ShapeDtypeStruct((N_PAD, HID), jnp.float32),
    )(x, W1)

    deg_parts = _deg_pass(ei_r, ew_r)

    acc1, dis = _edge_pass1(ei_r, ew_p, deg_parts, h1)

    acc2, x1p = _edge_pass2(ei_r, ew_p, deg_parts, h1, acc1, b1)

    out, x1 = pl.pallas_call(
        _final_body,
        grid=(N_NODES // _FBLK,),
        in_specs=[
            pl.BlockSpec((NC, _FBLK, HID), lambda i: (0, i, 0)),
            pl.BlockSpec((_FBLK, HID), lambda i: (i, 0)),
            pl.BlockSpec((_FBLK, 1), lambda i: (i, 0)),
            pl.BlockSpec((HID, N_CLS), lambda i: (0, 0)),
            pl.BlockSpec((1, N_CLS), lambda i: (0, 0)),
        ],
        out_specs=(
            pl.BlockSpec((_FBLK, N_CLS), lambda i: (i, 0)),
            pl.BlockSpec((_FBLK, HID), lambda i: (i, 0)),
        ),
        out_shape=(
            jax.ShapeDtypeStruct((N_NODES, N_CLS), jnp.float32),
            jax.ShapeDtypeStruct((N_NODES, HID), jnp.float32),
        ),
    )(acc2, x1p, dis[:, None], W2, b2[None, :])

    return (out, x1)


# speedup vs baseline: 1.3434x; 1.0260x over previous
import functools

import jax
import jax.numpy as jnp
from jax import lax
from jax.experimental import pallas as pl
from jax.experimental.pallas import tpu as pltpu
from jax.experimental.pallas import tpu_sc as plsc

N_NODES = 10000
N_PAD = 10240
E_EDGES = 320000
E_PAD = 327680
D_IN = 128
HID = 16
N_CLS = 40

NC = 2
NS = 16
NW = NC * NS
EPW = E_PAD // NW
SUB = 16
SUBE = 128
CHUNK = SUB * SUBE
NCHUNK = EPW // CHUNK
RPT = N_PAD // NS

_mesh = plsc.VectorSubcoreMesh(core_axis_name="c", subcore_axis_name="s")


def _rsqrt16(d):
    i = lax.bitcast_convert_type(d, jnp.int32)
    i = 0x5F3759DF - lax.shift_right_logical(i, 1)
    y = lax.bitcast_convert_type(i, jnp.float32)
    for _ in range(3):
        y = y * (1.5 - 0.5 * d * y * y)
    return y


@functools.partial(
    pl.kernel,
    mesh=_mesh,
    compiler_params=pltpu.CompilerParams(use_tc_tiling_on_sc=False),
    out_type=jax.ShapeDtypeStruct((NC, N_PAD), jnp.float32),
    scratch_types=[
        pltpu.VMEM((2, SUB, SUBE), jnp.int32),
        pltpu.VMEM((2, SUB, SUBE), jnp.float32),
        pltpu.VMEM((RPT,), jnp.float32),
        pltpu.VMEM_SHARED((N_PAD,), jnp.float32),
        pltpu.SemaphoreType.DMA,
        pltpu.SemaphoreType.DMA,
        pltpu.SemaphoreType.DMA,
        pltpu.SemaphoreType.DMA,
    ],
)
def _deg_pass(ei_hbm, ew_hbm, out_hbm, cidx, ewv, stage, acc,
              ds0, ds1, is0, is1):
    c = lax.axis_index("c")
    s = lax.axis_index("s")
    wid = s * NC + c
    dsem = (ds0, ds1)
    isem = (is0, is1)

    def _zero(i, _):
        stage[pl.ds(i * 16, 16)] = jnp.zeros((16,), jnp.float32)
        return 0

    lax.fori_loop(0, RPT // 16, _zero, 0)
    pltpu.sync_copy(stage, acc.at[pl.ds(s * RPT, RPT)])
    plsc.subcore_barrier()

    def _load(t):
        p = t % 2
        base = wid * (EPW // SUBE) + t * SUB
        return [
            pltpu.async_copy(ei_hbm.at[1, pl.ds(base, SUB)], cidx.at[p],
                             isem[p]),
            pltpu.async_copy(ew_hbm.at[pl.ds(base, SUB)], ewv.at[p],
                             isem[p]),
        ]

    def _fire(t):
        p = t % 2
        return [
            pltpu.async_copy(ewv.at[p, j], acc.at[cidx.at[p, j]],
                             dsem[p], add=True)
            for j in range(SUB)
        ]

    i_pend = {0: _load(0)}
    pend = {}
    for t in range(NCHUNK):
        for d in i_pend.pop(t):
            d.wait()
        if t + 1 < NCHUNK:
            if t - 1 in pend:
                for d in pend.pop(t - 1):
                    d.wait()
            i_pend[t + 1] = _load(t + 1)
        pend[t] = _fire(t)
    for t in sorted(pend):
        for d in pend.pop(t):
            d.wait()
    plsc.subcore_barrier()
    pltpu.sync_copy(acc.at[pl.ds(s * RPT, RPT)],
                    out_hbm.at[c, pl.ds(s * RPT, RPT)])


def _make_edge_pass(layer):
    extra_out = jax.ShapeDtypeStruct(
        (N_PAD,) if layer == 1 else (N_PAD, HID), jnp.float32)
    extra_scratch = [] if layer == 1 else [
        pltpu.VMEM((RPT, HID), jnp.float32),
        pltpu.VMEM((RPT, HID), jnp.float32),
        pltpu.VMEM((16,), jnp.float32),
    ]

    @functools.partial(
        pl.kernel,
        mesh=_mesh,
        compiler_params=pltpu.CompilerParams(use_tc_tiling_on_sc=False),
        out_type=(
            jax.ShapeDtypeStruct((NC, N_PAD, HID), jnp.float32),
            extra_out,
        ),
        scratch_types=[
            pltpu.VMEM((2, SUB, SUBE), jnp.int32),
            pltpu.VMEM((2, SUB, SUBE), jnp.int32),
            pltpu.VMEM((2, CHUNK), jnp.float32),
            pltpu.VMEM((2, CHUNK, HID), jnp.float32),
            pltpu.VMEM((RPT,), jnp.float32),
            pltpu.VMEM((RPT,), jnp.float32),
            pltpu.VMEM((RPT, HID), jnp.float32),
            pltpu.VMEM_SHARED((N_PAD, HID), jnp.float32),
            pltpu.VMEM_SHARED((N_PAD, HID), jnp.float32),
            pltpu.SemaphoreType.DMA,
            pltpu.SemaphoreType.DMA,
            pltpu.SemaphoreType.DMA,
            pltpu.SemaphoreType.DMA,
            pltpu.SemaphoreType.DMA,
            pltpu.SemaphoreType.DMA,
        ] + extra_scratch,
    )
    def _pass(*args):
        if layer == 1:
            (ei_hbm, ewf_hbm, deg_hbm, h1_hbm,
             acc_out, extra_hbm,
             ridx, cidx, ewf, buf, dv, d1v, hs, table, acc,
             gs0, gs1, ss0, ss1, is0, is1) = args
        else:
            (ei_hbm, ewf_hbm, deg_hbm, h1_hbm, acc1_hbm, b1_hbm,
             acc_out, extra_hbm,
             ridx, cidx, ewf, buf, dv, d1v, hs, table, acc,
             gs0, gs1, ss0, ss1, is0, is1, a0v, a1v, b1v) = args
        c = lax.axis_index("c")
        s = lax.axis_index("s")
        wid = s * NC + c
        gsem = (gs0, gs1)
        ssem = (ss0, ss1)
        isem = (is0, is1)
        base = s * RPT

        pltpu.sync_copy(deg_hbm.at[0, pl.ds(base, RPT)], dv)
        pltpu.sync_copy(deg_hbm.at[1, pl.ds(base, RPT)], d1v)
        pltpu.sync_copy(h1_hbm.at[pl.ds(base, RPT)], hs)
        if layer == 2:
            pltpu.sync_copy(acc1_hbm.at[0, pl.ds(base, RPT)], a0v)
            pltpu.sync_copy(acc1_hbm.at[1, pl.ds(base, RPT)], a1v)
            pltpu.sync_copy(b1_hbm, b1v)

        @plsc.parallel_loop(0, RPT // 16, unroll=2)
        def _dis(i):
            d = dv[pl.ds(i * 16, 16)] + d1v[pl.ds(i * 16, 16)] + 1.0
            dv[pl.ds(i * 16, 16)] = _rsqrt16(d)
        if layer == 1:
            pltpu.sync_copy(dv, extra_hbm.at[pl.ds(base, RPT)])

            def _grow(g):
                w = dv[pl.ds(g * 16, 16)]
                for k in range(16):
                    hs[g * 16 + k] = hs[g * 16 + k] * w[k]
        else:
            b1row = b1v[...]

            def _grow(g):
                w = dv[pl.ds(g * 16, 16)]
                for k in range(16):
                    r = g * 16 + k
                    x1 = w[k] * (a0v[r] + a1v[r]) \
                        + (w[k] * w[k]) * hs[r] + b1row
                    a0v[r] = x1
                    hs[r] = w[k] * x1

        plsc.parallel_loop(0, RPT // 16, unroll=2)(_grow)
        if layer == 2:
            pltpu.sync_copy(a0v, extra_hbm.at[pl.ds(base, RPT)])
        pltpu.sync_copy(hs, table.at[pl.ds(base, RPT)])

        @plsc.parallel_loop(0, RPT, unroll=4)
        def _zrow(i):
            hs[i] = jnp.zeros((HID,), jnp.float32)
        pltpu.sync_copy(hs, acc.at[pl.ds(base, RPT)])
        plsc.subcore_barrier()

        def _idx_fire(t):
            p = t % 2
            b2 = wid * (EPW // SUBE) + t * SUB
            return [
                pltpu.async_copy(ei_hbm.at[0, pl.ds(b2, SUB)], ridx.at[p],
                                 isem[p]),
                pltpu.async_copy(ei_hbm.at[1, pl.ds(b2, SUB)], cidx.at[p],
                                 isem[p]),
                pltpu.async_copy(
                    ewf_hbm.at[pl.ds(wid * EPW + t * CHUNK, CHUNK)],
                    ewf.at[p], isem[p]),
            ]

        def _fire_gathers(t):
            p = t % 2
            return [
                pltpu.async_copy(table.at[ridx.at[p, j]],
                                 buf.at[p, pl.ds(j * SUBE, SUBE)], gsem[p])
                for j in range(SUB)
            ]

        def _fire_scatters(t):
            p = t % 2
            return [
                pltpu.async_copy(buf.at[p, pl.ds(j * SUBE, SUBE)],
                                 acc.at[cidx.at[p, j]], ssem[p], add=True)
                for j in range(SUB)
            ]

        def _scale(t):
            p = t % 2

            @plsc.parallel_loop(0, CHUNK // 16, unroll=2)
            def _grp(g):
                w = ewf[p, pl.ds(g * 16, 16)]
                for k in range(16):
                    buf[p, g * 16 + k] = buf[p, g * 16 + k] * w[k]

        i_pend = {0: _idx_fire(0)}
        for d in i_pend.pop(0):
            d.wait()
        g_pend = {0: _fire_gathers(0)}
        s_pend = {}
        for t in range(NCHUNK):
            if t + 1 < NCHUNK:
                if t - 1 in s_pend:
                    for d in s_pend.pop(t - 1):
                        d.wait()
                i_pend[t + 1] = _idx_fire(t + 1)
            for d in g_pend.pop(t):
                d.wait()
            if t + 1 < NCHUNK:
                for d in i_pend.pop(t + 1):
                    d.wait()
                g_pend[t + 1] = _fire_gathers(t + 1)
            _scale(t)
            s_pend[t] = _fire_scatters(t)
        for t in sorted(s_pend):
            for d in s_pend.pop(t):
                d.wait()
        plsc.subcore_barrier()
        pltpu.sync_copy(acc.at[pl.ds(base, RPT)],
                        acc_out.at[c, pl.ds(base, RPT)])

    return _pass


_edge_pass1 = _make_edge_pass(1)
_edge_pass2 = _make_edge_pass(2)


def _mm1_body(x_ref, w_ref, o_ref):
    o_ref[pl.ds(0, N_NODES), :] = jnp.dot(
        x_ref[...], w_ref[...], preferred_element_type=jnp.float32)
    o_ref[pl.ds(N_NODES, N_PAD - N_NODES), :] = jnp.zeros(
        (N_PAD - N_NODES, HID), jnp.float32)


def _final_body(acc2_ref, x1p_ref, dis_ref, w2_ref, b2_ref,
                o_ref, x1_ref):
    dis = dis_ref[...]
    x1 = x1p_ref[...]
    x1_ref[...] = x1
    agg = dis * (acc2_ref[0] + acc2_ref[1]) + (dis * dis) * x1
    x2 = jnp.dot(agg, w2_ref[...], preferred_element_type=jnp.float32) \
        + b2_ref[...]
    m = jnp.max(x2, axis=1, keepdims=True)
    e = jnp.exp(x2 - m)
    lse = jnp.log(jnp.sum(e, axis=1, keepdims=True))
    o_ref[...] = x2 - m - lse


_FBLK = 2000


def kernel(x, edge_index, edge_weight, W1, b1, W2, b2):
    ei_p = jnp.pad(edge_index, ((0, 0), (0, E_PAD - E_EDGES)))
    ew_p = jnp.pad(edge_weight, (0, E_PAD - E_EDGES))
    ei_r = ei_p.reshape(2, E_PAD // SUBE, SUBE)
    ew_r = ew_p.reshape(E_PAD // SUBE, SUBE)

    h1 = pl.pallas_call(
        _mm1_body,
        out_shape=jax.ShapeDtypeStruct((N_PAD, HID), jnp.float32),
    )(x, W1)

    deg_parts = _deg_pass(ei_r, ew_r)

    acc1, dis = _edge_pass1(ei_r, ew_p, deg_parts, h1)

    acc2, x1p = _edge_pass2(ei_r, ew_p, deg_parts, h1, acc1, b1)

    out, x1 = pl.pallas_call(
        _final_body,
        grid=(N_NODES // _FBLK,),
        in_specs=[
            pl.BlockSpec((NC, _FBLK, HID), lambda i: (0, i, 0)),
            pl.BlockSpec((_FBLK, HID), lambda i: (i, 0)),
            pl.BlockSpec((_FBLK, 1), lambda i: (i, 0)),
            pl.BlockSpec((HID, N_CLS), lambda i: (0, 0)),
            pl.BlockSpec((1, N_CLS), lambda i: (0, 0)),
        ],
        out_specs=(
            pl.BlockSpec((_FBLK, N_CLS), lambda i: (i, 0)),
            pl.BlockSpec((_FBLK, HID), lambda i: (i, 0)),
        ),
        out_shape=(
            jax.ShapeDtypeStruct((N_NODES, N_CLS), jnp.float32),
            jax.ShapeDtypeStruct((N_NODES, HID), jnp.float32),
        ),
    )(acc2, x1p, dis[:, None], W2, b2[None, :])

    return (out, x1)


# speedup vs baseline: 1.3723x; 1.0215x over previous
import functools

import jax
import jax.numpy as jnp
from jax import lax
from jax.experimental import pallas as pl
from jax.experimental.pallas import tpu as pltpu
from jax.experimental.pallas import tpu_sc as plsc

N_NODES = 10000
N_PAD = 10240
E_EDGES = 320000
E_PAD = 327680
D_IN = 128
HID = 16
N_CLS = 40

NC = 2
NS = 16
NW = NC * NS
EPW = E_PAD // NW
SUB = 16
SUBE = 128
CHUNK = SUB * SUBE
NCHUNK = EPW // CHUNK
RPT = N_PAD // NS

_mesh = plsc.VectorSubcoreMesh(core_axis_name="c", subcore_axis_name="s")


def _rsqrt16(d):
    i = lax.bitcast_convert_type(d, jnp.int32)
    i = 0x5F3759DF - lax.shift_right_logical(i, 1)
    y = lax.bitcast_convert_type(i, jnp.float32)
    for _ in range(3):
        y = y * (1.5 - 0.5 * d * y * y)
    return y


@functools.partial(
    pl.kernel,
    mesh=_mesh,
    compiler_params=pltpu.CompilerParams(use_tc_tiling_on_sc=False),
    out_type=jax.ShapeDtypeStruct((NC, N_PAD), jnp.float32),
    scratch_types=[
        pltpu.VMEM((2, SUB, SUBE), jnp.int32),
        pltpu.VMEM((2, SUB, SUBE), jnp.float32),
        pltpu.VMEM((RPT,), jnp.float32),
        pltpu.VMEM_SHARED((N_PAD,), jnp.float32),
        pltpu.SemaphoreType.DMA,
        pltpu.SemaphoreType.DMA,
        pltpu.SemaphoreType.DMA,
        pltpu.SemaphoreType.DMA,
    ],
)
def _deg_pass(ei_hbm, ew_hbm, out_hbm, cidx, ewv, stage, acc,
              ds0, ds1, is0, is1):
    c = lax.axis_index("c")
    s = lax.axis_index("s")
    wid = s * NC + c
    dsem = (ds0, ds1)
    isem = (is0, is1)

    def _zero(i, _):
        stage[pl.ds(i * 16, 16)] = jnp.zeros((16,), jnp.float32)
        return 0

    lax.fori_loop(0, RPT // 16, _zero, 0)
    pltpu.sync_copy(stage, acc.at[pl.ds(s * RPT, RPT)])
    plsc.subcore_barrier()

    def _load(t):
        p = t % 2
        base = wid * (EPW // SUBE) + t * SUB
        return [
            pltpu.async_copy(ei_hbm.at[1, pl.ds(base, SUB)], cidx.at[p],
                             isem[p]),
            pltpu.async_copy(ew_hbm.at[pl.ds(base, SUB)], ewv.at[p],
                             isem[p]),
        ]

    def _fire(t):
        p = t % 2
        return [
            pltpu.async_copy(ewv.at[p, j], acc.at[cidx.at[p, j]],
                             dsem[p], add=True)
            for j in range(SUB)
        ]

    i_pend = {0: _load(0)}
    pend = {}
    for t in range(NCHUNK):
        for d in i_pend.pop(t):
            d.wait()
        if t + 1 < NCHUNK:
            if t - 1 in pend:
                for d in pend.pop(t - 1):
                    d.wait()
            i_pend[t + 1] = _load(t + 1)
        pend[t] = _fire(t)
    for t in sorted(pend):
        for d in pend.pop(t):
            d.wait()
    plsc.subcore_barrier()
    pltpu.sync_copy(acc.at[pl.ds(s * RPT, RPT)],
                    out_hbm.at[c, pl.ds(s * RPT, RPT)])


def _make_edge_pass(layer):
    extra_out = jax.ShapeDtypeStruct(
        (N_PAD,) if layer == 1 else (N_PAD, HID), jnp.float32)
    extra_scratch = [] if layer == 1 else [
        pltpu.VMEM((RPT, HID), jnp.float32),
        pltpu.VMEM((RPT, HID), jnp.float32),
        pltpu.VMEM((16,), jnp.float32),
    ]

    @functools.partial(
        pl.kernel,
        mesh=_mesh,
        compiler_params=pltpu.CompilerParams(use_tc_tiling_on_sc=False),
        out_type=(
            jax.ShapeDtypeStruct((NC, N_PAD, HID), jnp.float32),
            extra_out,
        ),
        scratch_types=[
            pltpu.VMEM((2, SUB, SUBE), jnp.int32),
            pltpu.VMEM((2, SUB, SUBE), jnp.int32),
            pltpu.VMEM((2, CHUNK), jnp.float32),
            pltpu.VMEM((2, CHUNK, HID), jnp.float32),
            pltpu.VMEM((RPT,), jnp.float32),
            pltpu.VMEM((RPT,), jnp.float32),
            pltpu.VMEM((RPT, HID), jnp.float32),
            pltpu.VMEM_SHARED((N_PAD, HID), jnp.float32),
            pltpu.VMEM_SHARED((N_PAD, HID), jnp.float32),
            pltpu.SemaphoreType.DMA,
            pltpu.SemaphoreType.DMA,
            pltpu.SemaphoreType.DMA,
            pltpu.SemaphoreType.DMA,
            pltpu.SemaphoreType.DMA,
            pltpu.SemaphoreType.DMA,
        ] + extra_scratch,
    )
    def _pass(*args):
        if layer == 1:
            (ei_hbm, ewf_hbm, deg_hbm, h1_hbm,
             acc_out, extra_hbm,
             ridx, cidx, ewf, buf, dv, d1v, hs, table, acc,
             gs0, gs1, ss0, ss1, is0, is1) = args
        else:
            (ei_hbm, ewf_hbm, deg_hbm, h1_hbm, acc1_hbm, b1_hbm,
             acc_out, extra_hbm,
             ridx, cidx, ewf, buf, dv, d1v, hs, table, acc,
             gs0, gs1, ss0, ss1, is0, is1, a0v, a1v, b1v) = args
        c = lax.axis_index("c")
        s = lax.axis_index("s")
        wid = s * NC + c
        gsem = (gs0, gs1)
        ssem = (ss0, ss1)
        isem = (is0, is1)
        base = s * RPT

        pro = [
            pltpu.async_copy(deg_hbm.at[0, pl.ds(base, RPT)], dv, is0),
            pltpu.async_copy(deg_hbm.at[1, pl.ds(base, RPT)], d1v, is0),
            pltpu.async_copy(h1_hbm.at[pl.ds(base, RPT)], hs, is0),
        ]
        if layer == 2:
            pro += [
                pltpu.async_copy(acc1_hbm.at[0, pl.ds(base, RPT)], a0v, is0),
                pltpu.async_copy(acc1_hbm.at[1, pl.ds(base, RPT)], a1v, is0),
                pltpu.async_copy(b1_hbm, b1v, is0),
            ]
        for d in pro:
            d.wait()

        @plsc.parallel_loop(0, RPT // 16, unroll=2)
        def _dis(i):
            d = dv[pl.ds(i * 16, 16)] + d1v[pl.ds(i * 16, 16)] + 1.0
            dv[pl.ds(i * 16, 16)] = _rsqrt16(d)
        if layer == 1:
            pltpu.sync_copy(dv, extra_hbm.at[pl.ds(base, RPT)])

            def _grow(g):
                w = dv[pl.ds(g * 16, 16)]
                for k in range(16):
                    hs[g * 16 + k] = hs[g * 16 + k] * w[k]
        else:
            b1row = b1v[...]

            def _grow(g):
                w = dv[pl.ds(g * 16, 16)]
                for k in range(16):
                    r = g * 16 + k
                    x1 = w[k] * (a0v[r] + a1v[r]) \
                        + (w[k] * w[k]) * hs[r] + b1row
                    a0v[r] = x1
                    hs[r] = w[k] * x1

        plsc.parallel_loop(0, RPT // 16, unroll=2)(_grow)
        if layer == 2:
            @pl.when(c == 0)
            def _emit_x1():
                pltpu.sync_copy(a0v, extra_hbm.at[pl.ds(base, RPT)])
        pltpu.sync_copy(hs, table.at[pl.ds(base, RPT)])

        @plsc.parallel_loop(0, RPT, unroll=4)
        def _zrow(i):
            hs[i] = jnp.zeros((HID,), jnp.float32)
        pltpu.sync_copy(hs, acc.at[pl.ds(base, RPT)])
        plsc.subcore_barrier()

        def _idx_fire(t):
            p = t % 2
            b2 = wid * (EPW // SUBE) + t * SUB
            return [
                pltpu.async_copy(ei_hbm.at[0, pl.ds(b2, SUB)], ridx.at[p],
                                 isem[p]),
                pltpu.async_copy(ei_hbm.at[1, pl.ds(b2, SUB)], cidx.at[p],
                                 isem[p]),
                pltpu.async_copy(
                    ewf_hbm.at[pl.ds(wid * EPW + t * CHUNK, CHUNK)],
                    ewf.at[p], isem[p]),
            ]

        def _fire_gathers(t):
            p = t % 2
            return [
                pltpu.async_copy(table.at[ridx.at[p, j]],
                                 buf.at[p, pl.ds(j * SUBE, SUBE)], gsem[p])
                for j in range(SUB)
            ]

        def _fire_scatters(t):
            p = t % 2
            return [
                pltpu.async_copy(buf.at[p, pl.ds(j * SUBE, SUBE)],
                                 acc.at[cidx.at[p, j]], ssem[p], add=True)
                for j in range(SUB)
            ]

        def _scale(t):
            p = t % 2

            @plsc.parallel_loop(0, CHUNK // 16, unroll=2)
            def _grp(g):
                w = ewf[p, pl.ds(g * 16, 16)]
                for k in range(16):
                    buf[p, g * 16 + k] = buf[p, g * 16 + k] * w[k]

        i_pend = {0: _idx_fire(0)}
        for d in i_pend.pop(0):
            d.wait()
        g_pend = {0: _fire_gathers(0)}
        s_pend = {}
        for t in range(NCHUNK):
            if t + 1 < NCHUNK:
                if t - 1 in s_pend:
                    for d in s_pend.pop(t - 1):
                        d.wait()
                i_pend[t + 1] = _idx_fire(t + 1)
            for d in g_pend.pop(t):
                d.wait()
            if t + 1 < NCHUNK:
                for d in i_pend.pop(t + 1):
                    d.wait()
                g_pend[t + 1] = _fire_gathers(t + 1)
            _scale(t)
            s_pend[t] = _fire_scatters(t)
        for t in sorted(s_pend):
            for d in s_pend.pop(t):
                d.wait()
        plsc.subcore_barrier()
        pltpu.sync_copy(acc.at[pl.ds(base, RPT)],
                        acc_out.at[c, pl.ds(base, RPT)])

    return _pass


_edge_pass1 = _make_edge_pass(1)
_edge_pass2 = _make_edge_pass(2)


def _mm1_body(x_ref, w_ref, o_ref):
    o_ref[pl.ds(0, N_NODES), :] = jnp.dot(
        x_ref[...], w_ref[...], preferred_element_type=jnp.float32)
    o_ref[pl.ds(N_NODES, N_PAD - N_NODES), :] = jnp.zeros(
        (N_PAD - N_NODES, HID), jnp.float32)


def _final_body(acc2_ref, x1p_ref, dis_ref, w2_ref, b2_ref,
                o_ref, x1_ref):
    dis = dis_ref[...]
    x1 = x1p_ref[...]
    x1_ref[...] = x1
    agg = dis * (acc2_ref[0] + acc2_ref[1]) + (dis * dis) * x1
    x2 = jnp.dot(agg, w2_ref[...], preferred_element_type=jnp.float32) \
        + b2_ref[...]
    m = jnp.max(x2, axis=1, keepdims=True)
    e = jnp.exp(x2 - m)
    lse = jnp.log(jnp.sum(e, axis=1, keepdims=True))
    o_ref[...] = x2 - m - lse


_FBLK = 1000


def kernel(x, edge_index, edge_weight, W1, b1, W2, b2):
    ei_p = jnp.pad(edge_index, ((0, 0), (0, E_PAD - E_EDGES)))
    ew_p = jnp.pad(edge_weight, (0, E_PAD - E_EDGES))
    ei_r = ei_p.reshape(2, E_PAD // SUBE, SUBE)
    ew_r = ew_p.reshape(E_PAD // SUBE, SUBE)

    h1 = pl.pallas_call(
        _mm1_body,
        out_shape=jax.ShapeDtypeStruct((N_PAD, HID), jnp.float32),
    )(x, W1)

    deg_parts = _deg_pass(ei_r, ew_r)

    acc1, dis = _edge_pass1(ei_r, ew_p, deg_parts, h1)

    acc2, x1p = _edge_pass2(ei_r, ew_p, deg_parts, h1, acc1, b1)

    out, x1 = pl.pallas_call(
        _final_body,
        grid=(N_NODES // _FBLK,),
        in_specs=[
            pl.BlockSpec((NC, _FBLK, HID), lambda i: (0, i, 0)),
            pl.BlockSpec((_FBLK, HID), lambda i: (i, 0)),
            pl.BlockSpec((_FBLK, 1), lambda i: (i, 0)),
            pl.BlockSpec((HID, N_CLS), lambda i: (0, 0)),
            pl.BlockSpec((1, N_CLS), lambda i: (0, 0)),
        ],
        out_specs=(
            pl.BlockSpec((_FBLK, N_CLS), lambda i: (i, 0)),
            pl.BlockSpec((_FBLK, HID), lambda i: (i, 0)),
        ),
        out_shape=(
            jax.ShapeDtypeStruct((N_NODES, N_CLS), jnp.float32),
            jax.ShapeDtypeStruct((N_NODES, HID), jnp.float32),
        ),
    )(acc2, x1p, dis[:, None], W2, b2[None, :])

    return (out, x1)


# speedup vs baseline: 1.3916x; 1.0141x over previous
import functools

import jax
import jax.numpy as jnp
from jax import lax
from jax.experimental import pallas as pl
from jax.experimental.pallas import tpu as pltpu
from jax.experimental.pallas import tpu_sc as plsc

N_NODES = 10000
N_PAD = 10240
E_EDGES = 320000
E_PAD = 327680
D_IN = 128
HID = 16
N_CLS = 40

NC = 2
NS = 16
NW = NC * NS
EPW = E_PAD // NW
SUB = 16
SUBE = 128
CHUNK = SUB * SUBE
NCHUNK = EPW // CHUNK
RPT = N_PAD // NS

_mesh = plsc.VectorSubcoreMesh(core_axis_name="c", subcore_axis_name="s")


def _rsqrt16(d):
    i = lax.bitcast_convert_type(d, jnp.int32)
    i = 0x5F3759DF - lax.shift_right_logical(i, 1)
    y = lax.bitcast_convert_type(i, jnp.float32)
    for _ in range(3):
        y = y * (1.5 - 0.5 * d * y * y)
    return y


@functools.partial(
    pl.kernel,
    mesh=_mesh,
    compiler_params=pltpu.CompilerParams(use_tc_tiling_on_sc=False),
    out_type=jax.ShapeDtypeStruct((NC, N_PAD), jnp.float32),
    scratch_types=[
        pltpu.VMEM((2, SUB, SUBE), jnp.int32),
        pltpu.VMEM((2, SUB, SUBE), jnp.float32),
        pltpu.VMEM((RPT,), jnp.float32),
        pltpu.VMEM_SHARED((N_PAD,), jnp.float32),
        pltpu.SemaphoreType.DMA,
        pltpu.SemaphoreType.DMA,
        pltpu.SemaphoreType.DMA,
        pltpu.SemaphoreType.DMA,
    ],
)
def _deg_pass(ei_hbm, ew_hbm, out_hbm, cidx, ewv, stage, acc,
              ds0, ds1, is0, is1):
    c = lax.axis_index("c")
    s = lax.axis_index("s")
    wid = s * NC + c
    dsem = (ds0, ds1)
    isem = (is0, is1)

    def _zero(i, _):
        stage[pl.ds(i * 16, 16)] = jnp.zeros((16,), jnp.float32)
        return 0

    lax.fori_loop(0, RPT // 16, _zero, 0)
    pltpu.sync_copy(stage, acc.at[pl.ds(s * RPT, RPT)])
    plsc.subcore_barrier()

    def _load(t):
        p = t % 2
        base = wid * (EPW // SUBE) + t * SUB
        return [
            pltpu.async_copy(ei_hbm.at[1, pl.ds(base, SUB)], cidx.at[p],
                             isem[p]),
            pltpu.async_copy(ew_hbm.at[pl.ds(base, SUB)], ewv.at[p],
                             isem[p]),
        ]

    def _fire(t):
        p = t % 2
        return [
            pltpu.async_copy(ewv.at[p, j], acc.at[cidx.at[p, j]],
                             dsem[p], add=True)
            for j in range(SUB)
        ]

    i_pend = {0: _load(0)}
    pend = {}
    for t in range(NCHUNK):
        for d in i_pend.pop(t):
            d.wait()
        if t + 1 < NCHUNK:
            if t - 1 in pend:
                for d in pend.pop(t - 1):
                    d.wait()
            i_pend[t + 1] = _load(t + 1)
        pend[t] = _fire(t)
    for t in sorted(pend):
        for d in pend.pop(t):
            d.wait()
    plsc.subcore_barrier()
    pltpu.sync_copy(acc.at[pl.ds(s * RPT, RPT)],
                    out_hbm.at[c, pl.ds(s * RPT, RPT)])


def _make_edge_pass(layer):
    extra_out = jax.ShapeDtypeStruct(
        (N_PAD,) if layer == 1 else (N_PAD, HID), jnp.float32)
    extra_scratch = [] if layer == 1 else [
        pltpu.VMEM((RPT, HID), jnp.float32),
        pltpu.VMEM((RPT, HID), jnp.float32),
        pltpu.VMEM((16,), jnp.float32),
    ]

    @functools.partial(
        pl.kernel,
        mesh=_mesh,
        compiler_params=pltpu.CompilerParams(use_tc_tiling_on_sc=False),
        out_type=(
            jax.ShapeDtypeStruct((NC, N_PAD, HID), jnp.float32),
            extra_out,
        ),
        scratch_types=[
            pltpu.VMEM((2, SUB, SUBE), jnp.int32),
            pltpu.VMEM((2, SUB, SUBE), jnp.int32),
            pltpu.VMEM((2, CHUNK), jnp.float32),
            pltpu.VMEM((2, CHUNK, HID), jnp.float32),
            pltpu.VMEM((RPT,), jnp.float32),
            pltpu.VMEM((RPT,), jnp.float32),
            pltpu.VMEM((RPT, HID), jnp.float32),
            pltpu.VMEM_SHARED((N_PAD, HID), jnp.float32),
            pltpu.VMEM_SHARED((N_PAD, HID), jnp.float32),
            pltpu.SemaphoreType.DMA,
            pltpu.SemaphoreType.DMA,
            pltpu.SemaphoreType.DMA,
            pltpu.SemaphoreType.DMA,
            pltpu.SemaphoreType.DMA,
            pltpu.SemaphoreType.DMA,
        ] + extra_scratch,
    )
    def _pass(*args):
        if layer == 1:
            (ei_hbm, ewf_hbm, deg_hbm, h1_hbm,
             acc_out, extra_hbm,
             ridx, cidx, ewf, buf, dv, d1v, hs, table, acc,
             gs0, gs1, ss0, ss1, is0, is1) = args
        else:
            (ei_hbm, ewf_hbm, deg_hbm, h1_hbm, acc1_hbm, b1_hbm,
             acc_out, extra_hbm,
             ridx, cidx, ewf, buf, dv, d1v, hs, table, acc,
             gs0, gs1, ss0, ss1, is0, is1, a0v, a1v, b1v) = args
        c = lax.axis_index("c")
        s = lax.axis_index("s")
        wid = s * NC + c
        gsem = (gs0, gs1)
        ssem = (ss0, ss1)
        isem = (is0, is1)
        base = s * RPT

        pro = [
            pltpu.async_copy(deg_hbm.at[0, pl.ds(base, RPT)], dv, is0),
            pltpu.async_copy(deg_hbm.at[1, pl.ds(base, RPT)], d1v, is0),
            pltpu.async_copy(h1_hbm.at[pl.ds(base, RPT)], hs, is0),
        ]
        if layer == 2:
            pro += [
                pltpu.async_copy(acc1_hbm.at[0, pl.ds(base, RPT)], a0v, is0),
                pltpu.async_copy(acc1_hbm.at[1, pl.ds(base, RPT)], a1v, is0),
                pltpu.async_copy(b1_hbm, b1v, is0),
            ]
        for d in pro:
            d.wait()

        @plsc.parallel_loop(0, RPT // 16, unroll=2)
        def _dis(i):
            d = dv[pl.ds(i * 16, 16)] + d1v[pl.ds(i * 16, 16)] + 1.0
            dv[pl.ds(i * 16, 16)] = _rsqrt16(d)
        if layer == 1:
            pltpu.sync_copy(dv, extra_hbm.at[pl.ds(base, RPT)])

            def _grow(g):
                w = dv[pl.ds(g * 16, 16)]
                for k in range(16):
                    hs[g * 16 + k] = hs[g * 16 + k] * w[k]
        else:
            b1row = b1v[...]

            def _grow(g):
                w = dv[pl.ds(g * 16, 16)]
                for k in range(16):
                    r = g * 16 + k
                    x1 = w[k] * (a0v[r] + a1v[r]) \
                        + (w[k] * w[k]) * hs[r] + b1row
                    a0v[r] = x1
                    hs[r] = w[k] * x1

        plsc.parallel_loop(0, RPT // 16, unroll=2)(_grow)
        if layer == 2:
            @pl.when(c == 0)
            def _emit_x1():
                pltpu.sync_copy(a0v, extra_hbm.at[pl.ds(base, RPT)])
        pltpu.sync_copy(hs, table.at[pl.ds(base, RPT)])

        @plsc.parallel_loop(0, RPT, unroll=4)
        def _zrow(i):
            hs[i] = jnp.zeros((HID,), jnp.float32)
        pltpu.sync_copy(hs, acc.at[pl.ds(base, RPT)])
        plsc.subcore_barrier()

        def _idx_fire(t):
            p = t % 2
            b2 = wid * (EPW // SUBE) + t * SUB
            return [
                pltpu.async_copy(ei_hbm.at[0, pl.ds(b2, SUB)], ridx.at[p],
                                 isem[p]),
                pltpu.async_copy(ei_hbm.at[1, pl.ds(b2, SUB)], cidx.at[p],
                                 isem[p]),
                pltpu.async_copy(
                    ewf_hbm.at[pl.ds(wid * EPW + t * CHUNK, CHUNK)],
                    ewf.at[p], isem[p]),
            ]

        def _fire_gathers(t):
            p = t % 2
            return [
                pltpu.async_copy(table.at[ridx.at[p, j]],
                                 buf.at[p, pl.ds(j * SUBE, SUBE)], gsem[p])
                for j in range(SUB)
            ]

        def _fire_scatters(t):
            p = t % 2
            return [
                pltpu.async_copy(buf.at[p, pl.ds(j * SUBE, SUBE)],
                                 acc.at[cidx.at[p, j]], ssem[p], add=True)
                for j in range(SUB)
            ]

        def _scale(t):
            p = t % 2

            @plsc.parallel_loop(0, CHUNK // 16, unroll=2)
            def _grp(g):
                w = ewf[p, pl.ds(g * 16, 16)]
                for k in range(16):
                    buf[p, g * 16 + k] = buf[p, g * 16 + k] * w[k]

        i_pend = {0: _idx_fire(0)}
        for d in i_pend.pop(0):
            d.wait()
        g_pend = {0: _fire_gathers(0)}
        s_pend = {}
        for t in range(NCHUNK):
            if t + 1 < NCHUNK:
                if t - 1 in s_pend:
                    for d in s_pend.pop(t - 1):
                        d.wait()
                i_pend[t + 1] = _idx_fire(t + 1)
            for d in g_pend.pop(t):
                d.wait()
            if t + 1 < NCHUNK:
                for d in i_pend.pop(t + 1):
                    d.wait()
                g_pend[t + 1] = _fire_gathers(t + 1)
            _scale(t)
            s_pend[t] = _fire_scatters(t)
        for t in sorted(s_pend):
            for d in s_pend.pop(t):
                d.wait()
        plsc.subcore_barrier()
        pltpu.sync_copy(acc.at[pl.ds(base, RPT)],
                        acc_out.at[c, pl.ds(base, RPT)])

    return _pass


_edge_pass1 = _make_edge_pass(1)
_edge_pass2 = _make_edge_pass(2)


def _mm1_body(x_ref, w_ref, o_ref):
    o_ref[pl.ds(0, N_NODES), :] = jnp.dot(
        x_ref[...], w_ref[...], preferred_element_type=jnp.float32)
    o_ref[pl.ds(N_NODES, N_PAD - N_NODES), :] = jnp.zeros(
        (N_PAD - N_NODES, HID), jnp.float32)


def _final_body(acc2_ref, x1p_ref, dis_ref, w2_ref, b2_ref,
                o_ref, x1_ref):
    dis = dis_ref[...]
    x1 = x1p_ref[...]
    x1_ref[...] = x1
    agg = dis * (acc2_ref[0] + acc2_ref[1]) + (dis * dis) * x1
    x2 = jnp.dot(agg, w2_ref[...], preferred_element_type=jnp.float32) \
        + b2_ref[...]
    m = jnp.max(x2, axis=1, keepdims=True)
    e = jnp.exp(x2 - m)
    lse = jnp.log(jnp.sum(e, axis=1, keepdims=True))
    o_ref[...] = x2 - m - lse


_FBLK = 2000


def kernel(x, edge_index, edge_weight, W1, b1, W2, b2):
    ei_p = jnp.pad(edge_index, ((0, 0), (0, E_PAD - E_EDGES)))
    ew_p = jnp.pad(edge_weight, (0, E_PAD - E_EDGES))
    ei_r = ei_p.reshape(2, E_PAD // SUBE, SUBE)
    ew_r = ew_p.reshape(E_PAD // SUBE, SUBE)

    h1 = pl.pallas_call(
        _mm1_body,
        out_shape=jax.ShapeDtypeStruct((N_PAD, HID), jnp.float32),
    )(x, W1)

    deg_parts = _deg_pass(ei_r, ew_r)

    acc1, dis = _edge_pass1(ei_r, ew_p, deg_parts, h1)

    acc2, x1p = _edge_pass2(ei_r, ew_p, deg_parts, h1, acc1, b1)

    out, x1 = pl.pallas_call(
        _final_body,
        grid=(N_NODES // _FBLK,),
        in_specs=[
            pl.BlockSpec((NC, _FBLK, HID), lambda i: (0, i, 0)),
            pl.BlockSpec((_FBLK, HID), lambda i: (i, 0)),
            pl.BlockSpec((_FBLK, 1), lambda i: (i, 0)),
            pl.BlockSpec((HID, N_CLS), lambda i: (0, 0)),
            pl.BlockSpec((1, N_CLS), lambda i: (0, 0)),
        ],
        out_specs=(
            pl.BlockSpec((_FBLK, N_CLS), lambda i: (i, 0)),
            pl.BlockSpec((_FBLK, HID), lambda i: (i, 0)),
        ),
        out_shape=(
            jax.ShapeDtypeStruct((N_NODES, N_CLS), jnp.float32),
            jax.ShapeDtypeStruct((N_NODES, HID), jnp.float32),
        ),
    )(acc2, x1p, dis[:, None], W2, b2[None, :])

    return (out, x1)
